# baseline (speedup 1.0000x reference)
import functools

import jax
import jax.numpy as jnp
from jax import lax
from jax.experimental import pallas as pl
from jax.experimental.pallas import tpu as pltpu
from jax.experimental.pallas import tpu_sc as plsc

_TRIM = 0.2

_ROWS = 8
_SC_ROWS = 32
_N = 512 * 512
_K_TRIM = int((1.0 - _TRIM) * _N)
_CH = 32768


def _tree_sum(x):
    ones8 = jnp.full((8, x.shape[0]), 0.125, jnp.float32)
    partial = jax.lax.dot_general(
        ones8, x, (((1,), (0,)), ((), ())),
        preferred_element_type=jnp.float32,
    )
    return jnp.sum(partial)


def _row_kernel(pred_ref, tgt_ref, out_ref, acc_ref):
    b = pl.program_id(0)
    nb = pl.num_programs(0)
    h = pred_ref.shape[1]
    w = pred_ref.shape[2]
    n = h * w

    @pl.when(b == 0)
    def _init():
        acc_ref[0] = 0.0

    loss_sum = acc_ref[0]
    for r in range(_ROWS):
        tgt = tgt_ref[r]

        cnt_neg = _tree_sum((tgt <= 0).astype(jnp.float32)).astype(jnp.int32)
        idx = jnp.minimum(cnt_neg + int((1.0 - _TRIM) * n), n - 1)

        base = pl.multiple_of((idx // w) & ~7, 8)
        p8 = pred_ref[r, pl.ds(base, 8), :]
        t8 = tgt_ref[r, pl.ds(base, 8), :]
        r8 = jnp.where(t8 > 0, jnp.abs(p8 - t8), jnp.zeros_like(p8))
        fl8 = (
            jax.lax.broadcasted_iota(jnp.int32, (8, w), 0) * w
            + jax.lax.broadcasted_iota(jnp.int32, (8, w), 1)
        )
        thr = jnp.sum(jnp.where(fl8 == idx - base * w, r8, jnp.zeros_like(r8)))

        pred = pred_ref[r]
        d = jnp.abs(pred - tgt)
        keep = jnp.logical_and(tgt > 0, d <= thr)
        s = _tree_sum(jnp.where(keep, d, jnp.zeros_like(d)))

        cnt_pos = n - cnt_neg
        valid = cnt_pos > 0
        denom = jnp.where(valid, 2 * cnt_pos, 1).astype(jnp.float32)
        loss_sum = loss_sum + jnp.where(valid, s / denom, 0.0)

    acc_ref[0] = loss_sum

    @pl.when(b == nb - 1)
    def _fin():
        out_ref[0] = loss_sum


def _tc_call(prediction, target, tc_rows):
    B, H, W = prediction.shape
    return pl.pallas_call(
        _row_kernel,
        grid=(tc_rows // _ROWS,),
        in_specs=[
            pl.BlockSpec((_ROWS, H, W), lambda b: (b, 0, 0)),
            pl.BlockSpec((_ROWS, H, W), lambda b: (b, 0, 0)),
        ],
        out_specs=pl.BlockSpec(memory_space=pltpu.SMEM),
        out_shape=jax.ShapeDtypeStruct((1,), jnp.float32),
        scratch_shapes=[pltpu.SMEM((1,), jnp.float32)],
    )(prediction, target)


def _sc_chunk_pass(buf_loads, n_vec, op):
    zero = jnp.zeros((16,), jnp.float32)

    def body(i, accs):
        base = pl.multiple_of(i * 64, 64)
        out = []
        for u in range(4):
            vals = [ld(pl.ds(base + u * 16, 16)) for ld in buf_loads]
            out.append(accs[u] + op(*vals))
        return tuple(out)

    accs = lax.fori_loop(0, n_vec // 4, body, (zero, zero, zero, zero))
    return accs[0] + accs[1] + accs[2] + accs[3]


def _lane_sum(vec, scratch):
    del scratch
    tot = vec[0]
    for i in range(1, 16):
        tot = tot + vec[i]
    return tot


def _sc_body(pred_hbm, tgt_hbm, out_hbm, tbuf, pbuf, obuf):
    cid = lax.axis_index("c")
    sid = lax.axis_index("s")
    wid = sid * 2 + cid
    row_off = (64 - _SC_ROWS + wid) * _N

    cnt16 = jnp.zeros((16,), jnp.float32)
    for c in range(_N // _CH):
        pltpu.sync_copy(tgt_hbm.at[pl.ds(row_off + c * _CH, _CH)], tbuf)
        cnt16 = cnt16 + _sc_chunk_pass(
            [lambda s: tbuf[s]],
            _CH // 16,
            lambda t: jnp.where(t <= 0.0, 1.0, 0.0).astype(jnp.float32),
        )
    cnt_neg = _lane_sum(cnt16, obuf).astype(jnp.int32)
    idx = jnp.minimum(cnt_neg + _K_TRIM, _N - 1)

    aligned = pl.multiple_of((row_off + idx) & ~15, 8)
    pltpu.sync_copy(pred_hbm.at[pl.ds(aligned, 16)], pbuf.at[pl.ds(0, 16)])
    pltpu.sync_copy(tgt_hbm.at[pl.ds(aligned, 16)], tbuf.at[pl.ds(0, 16)])
    p16 = pbuf[pl.ds(0, 16)]
    t16 = tbuf[pl.ds(0, 16)]
    d16 = jnp.where(t16 > 0.0, jnp.abs(p16 - t16), jnp.zeros_like(p16))
    lane = lax.iota(jnp.int32, 16)
    off = (row_off + idx) - aligned
    thr = _lane_sum(jnp.where(lane == off, d16, jnp.zeros_like(d16)), obuf)

    s16 = jnp.zeros((16,), jnp.float32)

    def keep_op(p, t):
        d = jnp.abs(p - t)
        k = jnp.logical_and(t > 0.0, d <= thr)
        return jnp.where(k, d, jnp.zeros_like(d))

    for c in range(_N // _CH):
        pltpu.sync_copy(pred_hbm.at[pl.ds(row_off + c * _CH, _CH)], pbuf)
        pltpu.sync_copy(tgt_hbm.at[pl.ds(row_off + c * _CH, _CH)], tbuf)
        s16 = s16 + _sc_chunk_pass(
            [lambda sl: pbuf[sl], lambda sl: tbuf[sl]], _CH // 16, keep_op
        )
    s = _lane_sum(s16, obuf)

    cnt_pos_v = jnp.zeros((16,), jnp.float32) + (_N - cnt_neg).astype(jnp.float32)
    s_v = jnp.zeros((16,), jnp.float32) + s
    denom_v = jnp.maximum(2.0 * cnt_pos_v, 1.0)
    gate_v = jnp.where(cnt_pos_v > 0.0, 1.0, 0.0).astype(jnp.float32)
    obuf[...] = (s_v / denom_v) * gate_v
    pltpu.sync_copy(obuf, out_hbm.at[pl.ds(wid * 16, 16)])


def _sc_call(pred_flat, tgt_flat):
    mesh = plsc.VectorSubcoreMesh(core_axis_name="c", subcore_axis_name="s")
    f = pl.kernel(
        _sc_body,
        mesh=mesh,
        out_type=jax.ShapeDtypeStruct((_SC_ROWS * 16,), jnp.float32),
        scratch_types=[
            pltpu.VMEM((_CH,), jnp.float32),
            pltpu.VMEM((_CH,), jnp.float32),
            pltpu.VMEM((16,), jnp.float32),
        ],
    )
    return f(pred_flat, tgt_flat)


@functools.partial(jax.jit, static_argnames=())
def kernel(prediction, target):
    B, H, W = prediction.shape
    tc_rows = B - _SC_ROWS
    tc_sum = _tc_call(prediction, target, tc_rows)[0]
    sc_out = _sc_call(prediction.reshape(-1), target.reshape(-1))
    sc_losses = sc_out.reshape(_SC_ROWS, 16)[:, 0]
    return (tc_sum + jnp.sum(sc_losses)) / B


# speedup vs baseline: 4.9560x; 4.9560x over previous
import functools

import jax
import jax.numpy as jnp
from jax.experimental import pallas as pl
from jax.experimental.pallas import tpu as pltpu

_TRIM = 0.2


_ROWS = 8


def _tree_sum(x):
    ones8 = jnp.full((8, x.shape[0]), 0.125, jnp.float32)
    partial = jax.lax.dot_general(
        ones8, x, (((1,), (0,)), ((), ())),
        preferred_element_type=jnp.float32,
    )
    return jnp.sum(partial)


def _row_kernel(pred_ref, tgt_ref, out_ref, acc_ref):
    b = pl.program_id(0)
    nb = pl.num_programs(0)
    h = pred_ref.shape[1]
    w = pred_ref.shape[2]
    n = h * w

    @pl.when(b == 0)
    def _init():
        acc_ref[0] = 0.0

    loss_sum = acc_ref[0]
    cnts = []
    for r in range(_ROWS):
        cnts.append(
            _tree_sum((tgt_ref[r] <= 0).astype(jnp.float32)).astype(jnp.int32)
        )

    fl8 = (
        jax.lax.broadcasted_iota(jnp.int32, (8, w), 0) * w
        + jax.lax.broadcasted_iota(jnp.int32, (8, w), 1)
    )
    thrs = []
    for r in range(_ROWS):
        idx = jnp.minimum(cnts[r] + int((1.0 - _TRIM) * n), n - 1)
        base = pl.multiple_of((idx // w) & ~7, 8)
        p8 = pred_ref[r, pl.ds(base, 8), :]
        t8 = tgt_ref[r, pl.ds(base, 8), :]
        r8 = jnp.where(t8 > 0, jnp.abs(p8 - t8), jnp.zeros_like(p8))
        thrs.append(
            jnp.sum(jnp.where(fl8 == idx - base * w, r8, jnp.zeros_like(r8)))
        )

    for r in range(_ROWS):
        tgt = tgt_ref[r]
        pred = pred_ref[r]
        d = jnp.abs(pred - tgt)
        keep = jnp.logical_and(tgt > 0, d <= thrs[r])
        s = _tree_sum(jnp.where(keep, d, jnp.zeros_like(d)))
        cnt_pos = n - cnts[r]
        valid = cnt_pos > 0
        denom = jnp.where(valid, 2 * cnt_pos, 1).astype(jnp.float32)
        loss_sum = loss_sum + jnp.where(valid, s / denom, 0.0)

    acc_ref[0] = loss_sum

    @pl.when(b == nb - 1)
    def _fin():
        out_ref[0] = loss_sum / (nb * _ROWS)


@functools.partial(jax.jit, static_argnames=())
def kernel(prediction, target):
    B, H, W = prediction.shape
    out = pl.pallas_call(
        _row_kernel,
        grid=(B // _ROWS,),
        in_specs=[
            pl.BlockSpec((_ROWS, H, W), lambda b: (b, 0, 0)),
            pl.BlockSpec((_ROWS, H, W), lambda b: (b, 0, 0)),
        ],
        out_specs=pl.BlockSpec(memory_space=pltpu.SMEM),
        out_shape=jax.ShapeDtypeStruct((1,), jnp.float32),
        scratch_shapes=[pltpu.SMEM((1,), jnp.float32)],
    )(prediction, target)
    return out[0]
